# baseline (speedup 1.0000x reference)
import functools

import jax
import jax.numpy as jnp
from jax import lax
from jax.experimental import pallas as pl
from jax.experimental.pallas import tpu as pltpu
from jax.experimental.pallas import tpu_sc as plsc

NBUF = 4
LOOKAHEAD = 2


@functools.lru_cache(maxsize=None)
def _make_transpose(vocab: int, d_model: int):
    info = plsc.get_sparse_core_info()
    nc, ns = info.num_cores, info.num_subcores
    nw = nc * ns
    blk = 128
    n_full = vocab // blk
    tail = vocab - n_full * blk
    per_w = n_full // nw
    extra = n_full - per_w * nw
    mesh = plsc.VectorSubcoreMesh(core_axis_name="c", subcore_axis_name="s")

    @functools.partial(
        pl.kernel,
        out_type=jax.ShapeDtypeStruct((vocab, 128), jnp.float32),
        mesh=mesh,
        scratch_types=[
            pltpu.VMEM((2, d_model, blk), jnp.float32),
            pltpu.VMEM((2, blk, 128), jnp.float32),
            [pltpu.SemaphoreType.DMA] * 2,
            [pltpu.SemaphoreType.DMA] * 2,
        ],
        compiler_params=pltpu.CompilerParams(needs_layout_passes=False),
    )
    def transpose(tt_hbm, tail_hbm, out_hbm, in_v, out_v, lsems, ssems):
        wid = lax.axis_index("s") * nc + lax.axis_index("c")
        sw = per_w * wid + jnp.minimum(wid, extra)
        nb = per_w + jnp.where(wid < extra, 1, 0)
        iota = lax.iota(jnp.int32, 16)

        def fire_load(k, s):
            v0 = (sw + k) * blk
            pltpu.async_copy(tt_hbm.at[:, pl.ds(v0, blk)], in_v.at[s], lsems[s])

        def wait_load(k, s):
            v0 = (sw + k) * blk
            pltpu.make_async_copy(
                tt_hbm.at[:, pl.ds(v0, blk)], in_v.at[s], lsems[s]
            ).wait()

        def fire_store(k, s):
            v0 = (sw + k) * blk
            pltpu.async_copy(out_v.at[s], out_hbm.at[pl.ds(v0, blk)], ssems[s])

        def wait_store(k, s):
            v0 = (sw + k) * blk
            pltpu.make_async_copy(
                out_v.at[s], out_hbm.at[pl.ds(v0, blk)], ssems[s]
            ).wait()

        def do_transpose(s):
            def tbody(v, carry):
                col = iota * 0 + v
                for j in range(d_model // 16):
                    vals = plsc.load_gather(in_v.at[s], [j * 16 + iota, col])
                    out_v[s, v, pl.ds(j * 16, 16)] = vals
                return carry

            lax.fori_loop(0, blk, tbody, 0)

        fire_load(0, 0)
        fire_load(1, 1)

        def pair(t, carry):
            for s in range(2):
                k = 2 * t + s
                wait_load(k, s)

                @pl.when(t >= 1)
                def _():
                    wait_store(k - 2, s)

                do_transpose(s)
                fire_store(k, s)

                @pl.when(k + 2 < per_w)
                def _():
                    fire_load(k + 2, s)

            return carry

        lax.fori_loop(0, per_w // 2, pair, 0)
        wait_store(per_w - 2, 0)
        wait_store(per_w - 1, 1)

        @pl.when(nb > per_w)
        def _():
            k = per_w
            fire_load(k, 0)
            wait_load(k, 0)
            do_transpose(0)
            fire_store(k, 0)
            wait_store(k, 0)

        if tail:
            @pl.when(wid == nw - 1)
            def _():
                t0 = n_full * blk
                dst_v = out_v.at[1].at[pl.ds(0, tail)]
                pltpu.async_copy(tail_hbm, dst_v, lsems[1])
                pltpu.make_async_copy(tail_hbm, dst_v, lsems[1]).wait()
                dst_h = out_hbm.at[pl.ds(t0, tail)]
                pltpu.async_copy(dst_v, dst_h, ssems[1])
                pltpu.make_async_copy(dst_v, dst_h, ssems[1]).wait()

    return transpose


@functools.lru_cache(maxsize=None)
def _make_lookup(b0: int, b1: int, d_model: int):
    info = plsc.get_sparse_core_info()
    nc, ns = info.num_cores, info.num_subcores
    nw = nc * ns
    n = b0 // nw
    assert n % NBUF == 0
    splits = []
    off = 0
    while off < b1:
        size = min(128, b1 - off)
        splits.append((off, size))
        off += size
    splits = tuple(splits)
    mesh = plsc.VectorSubcoreMesh(core_axis_name="c", subcore_axis_name="s")

    @functools.partial(
        pl.kernel,
        out_type=jax.ShapeDtypeStruct((b0, b1, 128), jnp.float32),
        mesh=mesh,
        scratch_types=[
            pltpu.VMEM((n, b1), jnp.int32),
            pltpu.VMEM((NBUF, b1, 128), jnp.float32),
            [pltpu.SemaphoreType.DMA] * NBUF,
            [pltpu.SemaphoreType.DMA] * NBUF,
        ],
        compiler_params=pltpu.CompilerParams(use_tc_tiling_on_sc=False),
    )
    def lookup(idx_hbm, table_hbm, out_hbm, idx_v, rows_v, gsems, ssems):
        wid = lax.axis_index("s") * nc + lax.axis_index("c")
        base = wid * n
        pltpu.sync_copy(idx_hbm.at[pl.ds(base, n)], idx_v)

        def fire_gather(c, b):
            for off, size in splits:
                pltpu.async_copy(
                    table_hbm.at[idx_v.at[c].at[pl.ds(off, size)]],
                    rows_v.at[b].at[pl.ds(off, size)],
                    gsems[b],
                )

        def wait_gather(c, b):
            for off, size in splits:
                pltpu.make_async_copy(
                    table_hbm.at[idx_v.at[c].at[pl.ds(off, size)]],
                    rows_v.at[b].at[pl.ds(off, size)],
                    gsems[b],
                ).wait()

        def fire_store(c, b):
            pltpu.async_copy(
                rows_v.at[b].at[:, pl.ds(0, d_model)],
                out_hbm.at[base + c, :, pl.ds(0, d_model)],
                ssems[b],
            )

        def wait_store(c, b):
            pltpu.make_async_copy(
                rows_v.at[b].at[:, pl.ds(0, d_model)],
                out_hbm.at[base + c, :, pl.ds(0, d_model)],
                ssems[b],
            ).wait()

        for b in range(NBUF):
            fire_gather(b, b)

        def group(t, carry):
            for b in range(NBUF):
                g = t * NBUF + b
                wait_gather(g, b)
                fire_store(g, b)
                h = g + LOOKAHEAD
                hb = (b + LOOKAHEAD) % NBUF

                @pl.when(jnp.logical_and(h >= NBUF, h < n))
                def _():
                    wait_store(h - NBUF, hb)
                    fire_gather(h, hb)

            return carry

        lax.fori_loop(0, n // NBUF, group, 0)

        for b in range(NBUF):
            c = n - NBUF + b
            wait_store(c, b)

    return lookup


def kernel(x, table):
    b0, b1 = x.shape
    idx = x.astype(jnp.int32)
    vocab, d_model = table.shape
    n_full = vocab // 128 * 128
    tail_p = jnp.pad(table[n_full:], ((0, 0), (0, 128 - d_model)))
    tpad = _make_transpose(vocab, d_model)(table.T, tail_p)
    out = _make_lookup(b0, b1, table.shape[1])(idx, tpad)
    return out[:, :, : table.shape[1]]


# speedup vs baseline: 1.0026x; 1.0026x over previous
import functools

import jax
import jax.numpy as jnp
from jax import lax
from jax.experimental import pallas as pl
from jax.experimental.pallas import tpu as pltpu
from jax.experimental.pallas import tpu_sc as plsc

NBUF = 4
LOOKAHEAD = 2


@functools.lru_cache(maxsize=None)
def _make_transpose(vocab: int, d_model: int):
    info = plsc.get_sparse_core_info()
    nc, ns = info.num_cores, info.num_subcores
    nw = nc * ns
    blk = 128
    n_full = vocab // blk
    tail = vocab - n_full * blk
    per_w = n_full // nw
    extra = n_full - per_w * nw
    mesh = plsc.VectorSubcoreMesh(core_axis_name="c", subcore_axis_name="s")

    @functools.partial(
        pl.kernel,
        out_type=jax.ShapeDtypeStruct((vocab, 128), jnp.float32),
        mesh=mesh,
        scratch_types=[
            pltpu.VMEM((2, d_model, blk), jnp.float32),
            pltpu.VMEM((2, blk, 128), jnp.float32),
            [pltpu.SemaphoreType.DMA] * 2,
            [pltpu.SemaphoreType.DMA] * 2,
        ],
        compiler_params=pltpu.CompilerParams(needs_layout_passes=False),
    )
    def transpose(tt_hbm, tail_hbm, out_hbm, in_v, out_v, lsems, ssems):
        wid = lax.axis_index("s") * nc + lax.axis_index("c")
        sw = per_w * wid + jnp.minimum(wid, extra)
        nb = per_w + jnp.where(wid < extra, 1, 0)
        iota = lax.iota(jnp.int32, 16)

        def fire_load(k, s):
            v0 = (sw + k) * blk
            pltpu.async_copy(tt_hbm.at[:, pl.ds(v0, blk)], in_v.at[s], lsems[s])

        def wait_load(k, s):
            v0 = (sw + k) * blk
            pltpu.make_async_copy(
                tt_hbm.at[:, pl.ds(v0, blk)], in_v.at[s], lsems[s]
            ).wait()

        def fire_store(k, s):
            v0 = (sw + k) * blk
            pltpu.async_copy(out_v.at[s], out_hbm.at[pl.ds(v0, blk)], ssems[s])

        def wait_store(k, s):
            v0 = (sw + k) * blk
            pltpu.make_async_copy(
                out_v.at[s], out_hbm.at[pl.ds(v0, blk)], ssems[s]
            ).wait()

        def do_transpose(s):
            def tbody(t, carry):
                v0 = t * 8
                for dv in range(8):
                    v = v0 + dv
                    col = iota * 0 + v
                    for j in range(d_model // 16):
                        vals = plsc.load_gather(
                            in_v.at[s], [j * 16 + iota, col]
                        )
                        out_v[s, v, pl.ds(j * 16, 16)] = vals
                return carry

            lax.fori_loop(0, blk // 8, tbody, 0)

        fire_load(0, 0)
        fire_load(1, 1)

        def pair(t, carry):
            for s in range(2):
                k = 2 * t + s
                wait_load(k, s)

                @pl.when(t >= 1)
                def _():
                    wait_store(k - 2, s)

                do_transpose(s)
                fire_store(k, s)

                @pl.when(k + 2 < per_w)
                def _():
                    fire_load(k + 2, s)

            return carry

        lax.fori_loop(0, per_w // 2, pair, 0)
        wait_store(per_w - 2, 0)
        wait_store(per_w - 1, 1)

        @pl.when(nb > per_w)
        def _():
            k = per_w
            fire_load(k, 0)
            wait_load(k, 0)
            do_transpose(0)
            fire_store(k, 0)
            wait_store(k, 0)

        if tail:
            @pl.when(wid == nw - 1)
            def _():
                t0 = n_full * blk
                dst_v = out_v.at[1].at[pl.ds(0, tail)]
                pltpu.async_copy(tail_hbm, dst_v, lsems[1])
                pltpu.make_async_copy(tail_hbm, dst_v, lsems[1]).wait()
                dst_h = out_hbm.at[pl.ds(t0, tail)]
                pltpu.async_copy(dst_v, dst_h, ssems[1])
                pltpu.make_async_copy(dst_v, dst_h, ssems[1]).wait()

    return transpose


@functools.lru_cache(maxsize=None)
def _make_lookup(b0: int, b1: int, d_model: int):
    info = plsc.get_sparse_core_info()
    nc, ns = info.num_cores, info.num_subcores
    nw = nc * ns
    n = b0 // nw
    assert n % NBUF == 0
    splits = []
    off = 0
    while off < b1:
        size = min(128, b1 - off)
        splits.append((off, size))
        off += size
    splits = tuple(splits)
    mesh = plsc.VectorSubcoreMesh(core_axis_name="c", subcore_axis_name="s")

    @functools.partial(
        pl.kernel,
        out_type=jax.ShapeDtypeStruct((b0, b1, 128), jnp.float32),
        mesh=mesh,
        scratch_types=[
            pltpu.VMEM((n, b1), jnp.int32),
            pltpu.VMEM((NBUF, b1, 128), jnp.float32),
            [pltpu.SemaphoreType.DMA] * NBUF,
            [pltpu.SemaphoreType.DMA] * NBUF,
        ],
        compiler_params=pltpu.CompilerParams(use_tc_tiling_on_sc=False),
    )
    def lookup(idx_hbm, table_hbm, out_hbm, idx_v, rows_v, gsems, ssems):
        wid = lax.axis_index("s") * nc + lax.axis_index("c")
        base = wid * n
        pltpu.sync_copy(idx_hbm.at[pl.ds(base, n)], idx_v)

        def fire_gather(c, b):
            for off, size in splits:
                pltpu.async_copy(
                    table_hbm.at[idx_v.at[c].at[pl.ds(off, size)]],
                    rows_v.at[b].at[pl.ds(off, size)],
                    gsems[b],
                )

        def wait_gather(c, b):
            for off, size in splits:
                pltpu.make_async_copy(
                    table_hbm.at[idx_v.at[c].at[pl.ds(off, size)]],
                    rows_v.at[b].at[pl.ds(off, size)],
                    gsems[b],
                ).wait()

        def fire_store(c, b):
            pltpu.async_copy(
                rows_v.at[b].at[:, pl.ds(0, d_model)],
                out_hbm.at[base + c, :, pl.ds(0, d_model)],
                ssems[b],
            )

        def wait_store(c, b):
            pltpu.make_async_copy(
                rows_v.at[b].at[:, pl.ds(0, d_model)],
                out_hbm.at[base + c, :, pl.ds(0, d_model)],
                ssems[b],
            ).wait()

        for b in range(NBUF):
            fire_gather(b, b)

        def group(t, carry):
            for b in range(NBUF):
                g = t * NBUF + b
                wait_gather(g, b)
                fire_store(g, b)
                h = g + LOOKAHEAD
                hb = (b + LOOKAHEAD) % NBUF

                @pl.when(jnp.logical_and(h >= NBUF, h < n))
                def _():
                    wait_store(h - NBUF, hb)
                    fire_gather(h, hb)

            return carry

        lax.fori_loop(0, n // NBUF, group, 0)

        for b in range(NBUF):
            c = n - NBUF + b
            wait_store(c, b)

    return lookup


def kernel(x, table):
    b0, b1 = x.shape
    idx = x.astype(jnp.int32)
    vocab, d_model = table.shape
    n_full = vocab // 128 * 128
    tail_p = jnp.pad(table[n_full:], ((0, 0), (0, 128 - d_model)))
    tpad = _make_transpose(vocab, d_model)(table.T, tail_p)
    out = _make_lookup(b0, b1, table.shape[1])(idx, tpad)
    return out[:, :, : table.shape[1]]


# speedup vs baseline: 2.0265x; 2.0211x over previous
import functools

import jax
import jax.numpy as jnp
from jax import lax
from jax.experimental import pallas as pl
from jax.experimental.pallas import tpu as pltpu
from jax.experimental.pallas import tpu_sc as plsc

NBUF = 4
LOOKAHEAD = 2


@functools.lru_cache(maxsize=None)
def _make_lookup(b0: int, b1: int, d_model: int):
    info = plsc.get_sparse_core_info()
    nc, ns = info.num_cores, info.num_subcores
    nw = nc * ns
    n = b0 // nw
    assert n % NBUF == 0
    splits = []
    off = 0
    while off < b1:
        size = min(128, b1 - off)
        splits.append((off, size))
        off += size
    splits = tuple(splits)
    mesh = plsc.VectorSubcoreMesh(core_axis_name="c", subcore_axis_name="s")

    @functools.partial(
        pl.kernel,
        out_type=jax.ShapeDtypeStruct((b0, b1, 128), jnp.float32),
        mesh=mesh,
        scratch_types=[
            pltpu.VMEM((n, b1), jnp.int32),
            pltpu.VMEM((NBUF, b1, 128), jnp.float32),
            [pltpu.SemaphoreType.DMA] * NBUF,
            [pltpu.SemaphoreType.DMA] * NBUF,
        ],
        compiler_params=pltpu.CompilerParams(use_tc_tiling_on_sc=False),
    )
    def lookup(idx_hbm, table_hbm, out_hbm, idx_v, rows_v, gsems, ssems):
        wid = lax.axis_index("s") * nc + lax.axis_index("c")
        base = wid * n
        pltpu.sync_copy(idx_hbm.at[pl.ds(base, n)], idx_v)

        def fire_gather(c, b):
            for off, size in splits:
                pltpu.async_copy(
                    table_hbm.at[idx_v.at[c].at[pl.ds(off, size)]],
                    rows_v.at[b].at[pl.ds(off, size)],
                    gsems[b],
                )

        def wait_gather(c, b):
            for off, size in splits:
                pltpu.make_async_copy(
                    table_hbm.at[idx_v.at[c].at[pl.ds(off, size)]],
                    rows_v.at[b].at[pl.ds(off, size)],
                    gsems[b],
                ).wait()

        def fire_store(c, b):
            pltpu.async_copy(
                rows_v.at[b].at[:, pl.ds(0, d_model)],
                out_hbm.at[base + c, :, pl.ds(0, d_model)],
                ssems[b],
            )

        def wait_store(c, b):
            pltpu.make_async_copy(
                rows_v.at[b].at[:, pl.ds(0, d_model)],
                out_hbm.at[base + c, :, pl.ds(0, d_model)],
                ssems[b],
            ).wait()

        for b in range(NBUF):
            fire_gather(b, b)

        def group(t, carry):
            for b in range(NBUF):
                g = t * NBUF + b
                wait_gather(g, b)
                fire_store(g, b)
                h = g + LOOKAHEAD
                hb = (b + LOOKAHEAD) % NBUF

                @pl.when(jnp.logical_and(h >= NBUF, h < n))
                def _():
                    wait_store(h - NBUF, hb)
                    fire_gather(h, hb)

            return carry

        lax.fori_loop(0, n // NBUF, group, 0)

        for b in range(NBUF):
            c = n - NBUF + b
            wait_store(c, b)

    return lookup


def kernel(x, table):
    b0, b1 = x.shape
    idx = x.astype(jnp.int32)
    tpad = jnp.pad(table, ((0, 0), (0, 128 - table.shape[1])))
    out = _make_lookup(b0, b1, table.shape[1])(idx, tpad)
    return out[:, :, : table.shape[1]]
